# baseline (speedup 1.0000x reference)
import functools

import jax
import jax.numpy as jnp
from jax.experimental import pallas as pl
from jax.experimental.pallas import tpu as pltpu

_NUM_CHUNKS = 4


def _dma_copy(x_ref, o_ref, *sems):
    rows = x_ref.shape[0]
    chunk = rows // _NUM_CHUNKS
    copies = [
        pltpu.make_async_copy(
            x_ref.at[pl.ds(i * chunk, chunk), :],
            o_ref.at[pl.ds(i * chunk, chunk), :],
            sems[i],
        )
        for i in range(_NUM_CHUNKS)
    ]
    for c in copies:
        c.start()
    for c in copies:
        c.wait()


def kernel(x, idx, label):
    del idx, label
    return pl.pallas_call(
        _dma_copy,
        out_shape=jax.ShapeDtypeStruct(x.shape, x.dtype),
        in_specs=[pl.BlockSpec(memory_space=pl.ANY)],
        out_specs=pl.BlockSpec(memory_space=pl.ANY),
        scratch_shapes=[pltpu.SemaphoreType.DMA] * _NUM_CHUNKS,
    )(x)


# speedup vs baseline: 25.9623x; 25.9623x over previous
import jax
import jax.numpy as jnp
from jax.experimental import pallas as pl

_ROWS_PER_BLOCK = 256


def _copy_block(x_ref, o_ref):
    o_ref[...] = x_ref[...]


def kernel(x, idx, label):
    del idx, label
    rows, cols = x.shape
    grid = rows // _ROWS_PER_BLOCK
    return pl.pallas_call(
        _copy_block,
        out_shape=jax.ShapeDtypeStruct(x.shape, x.dtype),
        grid=(grid,),
        in_specs=[pl.BlockSpec((_ROWS_PER_BLOCK, cols), lambda i: (i, 0))],
        out_specs=pl.BlockSpec((_ROWS_PER_BLOCK, cols), lambda i: (i, 0)),
    )(x)


# speedup vs baseline: 33.2631x; 1.2812x over previous
import jax
import jax.numpy as jnp
from jax.experimental import pallas as pl

_ROWS_PER_BLOCK = 512


def _copy_block(x_ref, o_ref):
    o_ref[...] = x_ref[...]


def kernel(x, idx, label):
    del idx, label
    rows, cols = x.shape
    grid = rows // _ROWS_PER_BLOCK
    return pl.pallas_call(
        _copy_block,
        out_shape=jax.ShapeDtypeStruct(x.shape, x.dtype),
        grid=(grid,),
        in_specs=[pl.BlockSpec((_ROWS_PER_BLOCK, cols), lambda i: (i, 0))],
        out_specs=pl.BlockSpec((_ROWS_PER_BLOCK, cols), lambda i: (i, 0)),
    )(x)
